# baseline (speedup 1.0000x reference)
import functools

import jax
import jax.numpy as jnp
from jax import lax
from jax.experimental import pallas as pl
from jax.experimental.pallas import tpu as pltpu
from jax.experimental.pallas import tpu_sc as plsc

NUM_TYPES = 8
DATA_SIZE = 10000
IN_DIM = 256
DIM = 128
BATCH = 16384

NUM_WORKERS = 32
PER_WORKER = BATCH // NUM_WORKERS
CHUNK = 128
NUM_CHUNKS = PER_WORKER // CHUNK

PROJ_BM = 2000
TC_BLOCK = 512
NUM_TC_BLOCKS = BATCH // TC_BLOCK


def _project_tables(tables, W, b):
    def body(tab_ref, w_ref, b_ref, o_ref):
        o_ref[0] = jnp.dot(tab_ref[0], w_ref[0],
                           preferred_element_type=jnp.float32) + b_ref[0]

    return pl.pallas_call(
        body,
        grid=(NUM_TYPES, DATA_SIZE // PROJ_BM),
        in_specs=[
            pl.BlockSpec((1, PROJ_BM, IN_DIM), lambda t, i: (t, i, 0)),
            pl.BlockSpec((1, IN_DIM, DIM), lambda t, i: (t, 0, 0)),
            pl.BlockSpec((1, 1, DIM), lambda t, i: (t, 0, 0)),
        ],
        out_specs=pl.BlockSpec((1, PROJ_BM, DIM), lambda t, i: (t, i, 0)),
        out_shape=jax.ShapeDtypeStruct((NUM_TYPES, DATA_SIZE, DIM),
                                       jnp.float32),
        compiler_params=pltpu.CompilerParams(
            dimension_semantics=("arbitrary", "arbitrary"),
        ),
    )(tables, W, b.reshape(NUM_TYPES, 1, DIM))


def _sc_route(codes, indices, lookup_table):
    mesh = plsc.VectorSubcoreMesh(core_axis_name="c", subcore_axis_name="s")

    @functools.partial(
        pl.kernel,
        out_type=(
            jax.ShapeDtypeStruct((BATCH,), jnp.int32),
            jax.ShapeDtypeStruct((BATCH,), jnp.int32),
            jax.ShapeDtypeStruct((BATCH, DIM), jnp.float32),
        ),
        mesh=mesh,
        scratch_types=[
            pltpu.VMEM((PER_WORKER,), jnp.int32),
            pltpu.VMEM((PER_WORKER,), jnp.int32),
            pltpu.VMEM((PER_WORKER,), jnp.int32),
            pltpu.VMEM((CHUNK, DIM), jnp.float32),
            pltpu.VMEM((CHUNK, DIM), jnp.float32),
            pltpu.SemaphoreType.DMA,
            pltpu.SemaphoreType.DMA,
            pltpu.SemaphoreType.DMA,
        ],
    )
    def sc_kernel(codes_hbm, idx_hbm, lut_hbm,
                  row_out, code_out, lb_out,
                  idx_v, c_v, row_v, lb_v0, lb_v1,
                  csem, gsem, wsem):
        wid = lax.axis_index("s") * 2 + lax.axis_index("c")
        base = wid * PER_WORKER
        pltpu.sync_copy(idx_hbm.at[pl.ds(base, PER_WORKER)], idx_v)
        code_gets = [
            pltpu.async_copy(
                codes_hbm.at[idx_v.at[pl.ds(k * CHUNK, CHUNK)]],
                c_v.at[pl.ds(k * CHUNK, CHUNK)], csem)
            for k in range(NUM_CHUNKS)
        ]
        lb_bufs = (lb_v0, lb_v1)

        def fire(k):
            return pltpu.async_copy(
                lut_hbm.at[idx_v.at[pl.ds(k * CHUNK, CHUNK)]],
                lb_bufs[k % 2], gsem)

        pend = fire(0)
        writes = [None] * NUM_CHUNKS
        for k in range(NUM_CHUNKS):
            if k >= 1:
                writes[k - 1].wait()
            nxt = fire(k + 1) if k + 1 < NUM_CHUNKS else None
            pend.wait()
            writes[k] = pltpu.async_copy(
                lb_bufs[k % 2], lb_out.at[pl.ds(base + k * CHUNK, CHUNK)],
                wsem)
            pend = nxt
        for cg in code_gets:
            cg.wait()

        @pl.loop(0, PER_WORKER, step=16)
        def _(i):
            row_v[pl.ds(i, 16)] = jnp.maximum(c_v[pl.ds(i, 16)], 0)

        cp = pltpu.async_copy(c_v, code_out.at[pl.ds(base, PER_WORKER)], csem)
        rp = pltpu.async_copy(row_v, row_out.at[pl.ds(base, PER_WORKER)], csem)
        writes[-1].wait()
        cp.wait()
        rp.wait()

    return sc_kernel(codes, indices, lookup_table)


def _sc_gather_proj(rows, proj_flat):
    mesh = plsc.VectorSubcoreMesh(core_axis_name="c", subcore_axis_name="s")

    @functools.partial(
        pl.kernel,
        out_type=jax.ShapeDtypeStruct((BATCH, DIM), jnp.float32),
        mesh=mesh,
        scratch_types=[
            pltpu.VMEM((PER_WORKER,), jnp.int32),
            pltpu.VMEM((CHUNK, DIM), jnp.float32),
            pltpu.VMEM((CHUNK, DIM), jnp.float32),
            pltpu.SemaphoreType.DMA,
            pltpu.SemaphoreType.DMA,
        ],
    )
    def sc_kernel(rows_hbm, proj_hbm, out_hbm, row_v, b0, b1, gsem, wsem):
        wid = lax.axis_index("s") * 2 + lax.axis_index("c")
        base = wid * PER_WORKER
        pltpu.sync_copy(rows_hbm.at[pl.ds(base, PER_WORKER)], row_v)
        bufs = (b0, b1)

        def fire(k):
            return pltpu.async_copy(
                proj_hbm.at[row_v.at[pl.ds(k * CHUNK, CHUNK)]],
                bufs[k % 2], gsem)

        pend = fire(0)
        writes = [None] * NUM_CHUNKS
        for k in range(NUM_CHUNKS):
            if k >= 1:
                writes[k - 1].wait()
            nxt = fire(k + 1) if k + 1 < NUM_CHUNKS else None
            pend.wait()
            writes[k] = pltpu.async_copy(
                bufs[k % 2], out_hbm.at[pl.ds(base + k * CHUNK, CHUNK)], wsem)
            pend = nxt
        writes[-1].wait()

    return sc_kernel(rows, proj_flat)


def _tc_combine(codes_b, prow, lb):
    def body(c_ref, p_ref, l_ref, o_ref):
        o_ref[...] = jnp.where(c_ref[...] >= 0, p_ref[...], l_ref[...])

    return pl.pallas_call(
        body,
        grid=(NUM_TC_BLOCKS,),
        in_specs=[
            pl.BlockSpec((TC_BLOCK, 1), lambda i: (i, 0)),
            pl.BlockSpec((TC_BLOCK, DIM), lambda i: (i, 0)),
            pl.BlockSpec((TC_BLOCK, DIM), lambda i: (i, 0)),
        ],
        out_specs=pl.BlockSpec((TC_BLOCK, DIM), lambda i: (i, 0)),
        out_shape=jax.ShapeDtypeStruct((BATCH, DIM), jnp.float32),
        compiler_params=pltpu.CompilerParams(
            dimension_semantics=("arbitrary",),
        ),
    )(codes_b, prow, lb)


def kernel(indices, entity_types, entity_data_idx, tables, W, b, lookup_table):
    indices = indices.astype(jnp.int32)
    entity_types = entity_types.astype(jnp.int32)
    entity_data_idx = entity_data_idx.astype(jnp.int32)
    codes = jnp.where(entity_types < NUM_TYPES,
                      entity_types * DATA_SIZE + entity_data_idx,
                      -1).astype(jnp.int32)

    proj = _project_tables(tables, W, b)
    proj_flat = proj.reshape(NUM_TYPES * DATA_SIZE, DIM)
    rows, codes_b, lb = _sc_route(codes, indices, lookup_table)
    prow = _sc_gather_proj(rows, proj_flat)
    return _tc_combine(codes_b.reshape(BATCH, 1), prow, lb)


# speedup vs baseline: 1.7196x; 1.7196x over previous
import functools

import jax
import jax.numpy as jnp
from jax import lax
from jax.experimental import pallas as pl
from jax.experimental.pallas import tpu as pltpu
from jax.experimental.pallas import tpu_sc as plsc

NUM_TYPES = 8
DATA_SIZE = 10000
IN_DIM = 256
DIM = 128
BATCH = 16384

NUM_WORKERS = 32
PER_WORKER = BATCH // NUM_WORKERS
CHUNK = 128
NUM_CHUNKS = PER_WORKER // CHUNK

PROJ_BM = 2000
TC_BLOCK = 512
NUM_TC_BLOCKS = BATCH // TC_BLOCK


def _project_tables(tables, W, b):
    def body(tab_ref, w_ref, b_ref, o_ref):
        o_ref[0] = jnp.dot(tab_ref[0], w_ref[0],
                           preferred_element_type=jnp.float32) + b_ref[0]

    return pl.pallas_call(
        body,
        grid=(NUM_TYPES, DATA_SIZE // PROJ_BM),
        in_specs=[
            pl.BlockSpec((1, PROJ_BM, IN_DIM), lambda t, i: (t, i, 0)),
            pl.BlockSpec((1, IN_DIM, DIM), lambda t, i: (t, 0, 0)),
            pl.BlockSpec((1, 1, DIM), lambda t, i: (t, 0, 0)),
        ],
        out_specs=pl.BlockSpec((1, PROJ_BM, DIM), lambda t, i: (t, i, 0)),
        out_shape=jax.ShapeDtypeStruct((NUM_TYPES, DATA_SIZE, DIM),
                                       jnp.float32),
        compiler_params=pltpu.CompilerParams(
            dimension_semantics=("arbitrary", "arbitrary"),
        ),
    )(tables, W, b.reshape(NUM_TYPES, 1, DIM))


def _sc_route(codes, indices, lookup_table):
    mesh = plsc.VectorSubcoreMesh(core_axis_name="c", subcore_axis_name="s")

    @functools.partial(
        pl.kernel,
        out_type=(
            jax.ShapeDtypeStruct((BATCH,), jnp.int32),
            jax.ShapeDtypeStruct((BATCH,), jnp.int32),
            jax.ShapeDtypeStruct((BATCH, DIM), jnp.float32),
        ),
        mesh=mesh,
        scratch_types=[
            pltpu.VMEM((PER_WORKER,), jnp.int32),
            pltpu.VMEM((PER_WORKER,), jnp.int32),
            pltpu.VMEM((PER_WORKER,), jnp.int32),
            pltpu.VMEM((CHUNK, DIM), jnp.float32),
            pltpu.VMEM((CHUNK, DIM), jnp.float32),
            pltpu.SemaphoreType.DMA,
            pltpu.SemaphoreType.DMA,
            pltpu.SemaphoreType.DMA,
        ],
    )
    def sc_kernel(codes_hbm, idx_hbm, lut_hbm,
                  row_out, code_out, lb_out,
                  idx_v, c_v, row_v, lb_v0, lb_v1,
                  csem, gsem, wsem):
        wid = lax.axis_index("s") * 2 + lax.axis_index("c")
        base = wid * PER_WORKER
        pltpu.sync_copy(idx_hbm.at[pl.ds(base, PER_WORKER)], idx_v)
        code_gets = [
            pltpu.async_copy(
                codes_hbm.at[idx_v.at[pl.ds(k * CHUNK, CHUNK)]],
                c_v.at[pl.ds(k * CHUNK, CHUNK)], csem)
            for k in range(NUM_CHUNKS)
        ]
        lb_bufs = (lb_v0, lb_v1)

        def fire(k):
            return pltpu.async_copy(
                lut_hbm.at[idx_v.at[pl.ds(k * CHUNK, CHUNK)]],
                lb_bufs[k % 2], gsem)

        pend = fire(0)
        writes = [None] * NUM_CHUNKS
        for k in range(NUM_CHUNKS):
            if k >= 1:
                writes[k - 1].wait()
            nxt = fire(k + 1) if k + 1 < NUM_CHUNKS else None
            pend.wait()
            writes[k] = pltpu.async_copy(
                lb_bufs[k % 2], lb_out.at[pl.ds(base + k * CHUNK, CHUNK)],
                wsem)
            pend = nxt
        for cg in code_gets:
            cg.wait()

        @pl.loop(0, PER_WORKER, step=16)
        def _(i):
            c = c_v[pl.ds(i, 16)]
            pos = lax.iota(jnp.int32, 16) + (base + i)
            row_v[pl.ds(i, 16)] = jnp.where(c >= 0, c, pos)

        cp = pltpu.async_copy(c_v, code_out.at[pl.ds(base, PER_WORKER)], csem)
        rp = pltpu.async_copy(row_v, row_out.at[pl.ds(base, PER_WORKER)], csem)
        writes[-1].wait()
        cp.wait()
        rp.wait()

    return sc_kernel(codes, indices, lookup_table)


def _sc_gather_proj(rows, proj_flat):
    mesh = plsc.VectorSubcoreMesh(core_axis_name="c", subcore_axis_name="s")

    @functools.partial(
        pl.kernel,
        out_type=jax.ShapeDtypeStruct((BATCH, DIM), jnp.float32),
        mesh=mesh,
        scratch_types=[
            pltpu.VMEM((PER_WORKER,), jnp.int32),
            pltpu.VMEM((CHUNK, DIM), jnp.float32),
            pltpu.VMEM((CHUNK, DIM), jnp.float32),
            pltpu.SemaphoreType.DMA,
            pltpu.SemaphoreType.DMA,
        ],
    )
    def sc_kernel(rows_hbm, proj_hbm, out_hbm, row_v, b0, b1, gsem, wsem):
        wid = lax.axis_index("s") * 2 + lax.axis_index("c")
        base = wid * PER_WORKER
        pltpu.sync_copy(rows_hbm.at[pl.ds(base, PER_WORKER)], row_v)
        bufs = (b0, b1)

        def fire(k):
            return pltpu.async_copy(
                proj_hbm.at[row_v.at[pl.ds(k * CHUNK, CHUNK)]],
                bufs[k % 2], gsem)

        pend = fire(0)
        writes = [None] * NUM_CHUNKS
        for k in range(NUM_CHUNKS):
            if k >= 1:
                writes[k - 1].wait()
            nxt = fire(k + 1) if k + 1 < NUM_CHUNKS else None
            pend.wait()
            writes[k] = pltpu.async_copy(
                bufs[k % 2], out_hbm.at[pl.ds(base + k * CHUNK, CHUNK)], wsem)
            pend = nxt
        writes[-1].wait()

    return sc_kernel(rows, proj_flat)


def _tc_combine(codes_b, prow, lb):
    def body(c_ref, p_ref, l_ref, o_ref):
        o_ref[...] = jnp.where(c_ref[...] >= 0, p_ref[...], l_ref[...])

    return pl.pallas_call(
        body,
        grid=(NUM_TC_BLOCKS,),
        in_specs=[
            pl.BlockSpec((TC_BLOCK, 1), lambda i: (i, 0)),
            pl.BlockSpec((TC_BLOCK, DIM), lambda i: (i, 0)),
            pl.BlockSpec((TC_BLOCK, DIM), lambda i: (i, 0)),
        ],
        out_specs=pl.BlockSpec((TC_BLOCK, DIM), lambda i: (i, 0)),
        out_shape=jax.ShapeDtypeStruct((BATCH, DIM), jnp.float32),
        compiler_params=pltpu.CompilerParams(
            dimension_semantics=("arbitrary",),
        ),
    )(codes_b, prow, lb)


def kernel(indices, entity_types, entity_data_idx, tables, W, b, lookup_table):
    indices = indices.astype(jnp.int32)
    entity_types = entity_types.astype(jnp.int32)
    entity_data_idx = entity_data_idx.astype(jnp.int32)
    codes = jnp.where(entity_types < NUM_TYPES,
                      entity_types * DATA_SIZE + entity_data_idx,
                      -1).astype(jnp.int32)

    proj = _project_tables(tables, W, b)
    proj_flat = proj.reshape(NUM_TYPES * DATA_SIZE, DIM)
    rows, codes_b, lb = _sc_route(codes, indices, lookup_table)
    prow = _sc_gather_proj(rows, proj_flat)
    return _tc_combine(codes_b.reshape(BATCH, 1), prow, lb)


# speedup vs baseline: 1.9186x; 1.1157x over previous
import functools

import jax
import jax.numpy as jnp
from jax import lax
from jax.experimental import pallas as pl
from jax.experimental.pallas import tpu as pltpu
from jax.experimental.pallas import tpu_sc as plsc

NUM_TYPES = 8
DATA_SIZE = 10000
IN_DIM = 256
DIM = 128
BATCH = 16384

NUM_WORKERS = 32
PER_WORKER = BATCH // NUM_WORKERS
CHUNK = 128
NUM_CHUNKS = PER_WORKER // CHUNK

PROJ_BM = 2000
TC_BLOCK = 2048
NUM_TC_BLOCKS = BATCH // TC_BLOCK


def _project_tables(tables, W, b):
    def body(tab_ref, w_ref, b_ref, o_ref):
        o_ref[0] = jnp.dot(tab_ref[0].astype(jnp.bfloat16),
                           w_ref[0].astype(jnp.bfloat16),
                           preferred_element_type=jnp.float32) + b_ref[0]

    return pl.pallas_call(
        body,
        grid=(NUM_TYPES, DATA_SIZE // PROJ_BM),
        in_specs=[
            pl.BlockSpec((1, PROJ_BM, IN_DIM), lambda t, i: (t, i, 0)),
            pl.BlockSpec((1, IN_DIM, DIM), lambda t, i: (t, 0, 0)),
            pl.BlockSpec((1, 1, DIM), lambda t, i: (t, 0, 0)),
        ],
        out_specs=pl.BlockSpec((1, PROJ_BM, DIM), lambda t, i: (t, i, 0)),
        out_shape=jax.ShapeDtypeStruct((NUM_TYPES, DATA_SIZE, DIM),
                                       jnp.float32),
        compiler_params=pltpu.CompilerParams(
            dimension_semantics=("arbitrary", "arbitrary"),
        ),
    )(tables, W, b.reshape(NUM_TYPES, 1, DIM))


def _sc_route(codes, indices, lookup_table):
    mesh = plsc.VectorSubcoreMesh(core_axis_name="c", subcore_axis_name="s")

    @functools.partial(
        pl.kernel,
        out_type=(
            jax.ShapeDtypeStruct((BATCH,), jnp.int32),
            jax.ShapeDtypeStruct((BATCH,), jnp.int32),
            jax.ShapeDtypeStruct((BATCH, DIM), jnp.float32),
        ),
        mesh=mesh,
        scratch_types=[
            pltpu.VMEM((PER_WORKER,), jnp.int32),
            pltpu.VMEM((PER_WORKER,), jnp.int32),
            pltpu.VMEM((PER_WORKER,), jnp.int32),
            pltpu.VMEM((CHUNK, DIM), jnp.float32),
            pltpu.VMEM((CHUNK, DIM), jnp.float32),
            pltpu.SemaphoreType.DMA,
            pltpu.SemaphoreType.DMA,
            pltpu.SemaphoreType.DMA,
        ],
    )
    def sc_kernel(codes_hbm, idx_hbm, lut_hbm,
                  row_out, code_out, lb_out,
                  idx_v, c_v, row_v, lb_v0, lb_v1,
                  csem, gsem, wsem):
        wid = lax.axis_index("s") * 2 + lax.axis_index("c")
        base = wid * PER_WORKER
        pltpu.sync_copy(idx_hbm.at[pl.ds(base, PER_WORKER)], idx_v)
        code_gets = [
            pltpu.async_copy(
                codes_hbm.at[idx_v.at[pl.ds(k * CHUNK, CHUNK)]],
                c_v.at[pl.ds(k * CHUNK, CHUNK)], csem)
            for k in range(NUM_CHUNKS)
        ]
        lb_bufs = (lb_v0, lb_v1)

        def fire(k):
            return pltpu.async_copy(
                lut_hbm.at[idx_v.at[pl.ds(k * CHUNK, CHUNK)]],
                lb_bufs[k % 2], gsem)

        pend = fire(0)
        writes = [None] * NUM_CHUNKS
        for k in range(NUM_CHUNKS):
            if k >= 1:
                writes[k - 1].wait()
            nxt = fire(k + 1) if k + 1 < NUM_CHUNKS else None
            pend.wait()
            writes[k] = pltpu.async_copy(
                lb_bufs[k % 2], lb_out.at[pl.ds(base + k * CHUNK, CHUNK)],
                wsem)
            pend = nxt
        for cg in code_gets:
            cg.wait()

        @pl.loop(0, PER_WORKER, step=16)
        def _(i):
            c = c_v[pl.ds(i, 16)]
            pos = lax.iota(jnp.int32, 16) + (base + i)
            row_v[pl.ds(i, 16)] = jnp.where(c >= 0, c, pos)

        cp = pltpu.async_copy(c_v, code_out.at[pl.ds(base, PER_WORKER)], csem)
        rp = pltpu.async_copy(row_v, row_out.at[pl.ds(base, PER_WORKER)], csem)
        writes[-1].wait()
        cp.wait()
        rp.wait()

    return sc_kernel(codes, indices, lookup_table)


def _sc_gather_proj(rows, proj_flat):
    mesh = plsc.VectorSubcoreMesh(core_axis_name="c", subcore_axis_name="s")

    @functools.partial(
        pl.kernel,
        out_type=jax.ShapeDtypeStruct((BATCH, DIM), jnp.float32),
        mesh=mesh,
        scratch_types=[
            pltpu.VMEM((PER_WORKER,), jnp.int32),
            pltpu.VMEM((CHUNK, DIM), jnp.float32),
            pltpu.VMEM((CHUNK, DIM), jnp.float32),
            pltpu.SemaphoreType.DMA,
            pltpu.SemaphoreType.DMA,
        ],
    )
    def sc_kernel(rows_hbm, proj_hbm, out_hbm, row_v, b0, b1, gsem, wsem):
        wid = lax.axis_index("s") * 2 + lax.axis_index("c")
        base = wid * PER_WORKER
        pltpu.sync_copy(rows_hbm.at[pl.ds(base, PER_WORKER)], row_v)
        bufs = (b0, b1)

        def fire(k):
            return pltpu.async_copy(
                proj_hbm.at[row_v.at[pl.ds(k * CHUNK, CHUNK)]],
                bufs[k % 2], gsem)

        pend = fire(0)
        writes = [None] * NUM_CHUNKS
        for k in range(NUM_CHUNKS):
            if k >= 1:
                writes[k - 1].wait()
            nxt = fire(k + 1) if k + 1 < NUM_CHUNKS else None
            pend.wait()
            writes[k] = pltpu.async_copy(
                bufs[k % 2], out_hbm.at[pl.ds(base + k * CHUNK, CHUNK)], wsem)
            pend = nxt
        writes[-1].wait()

    return sc_kernel(rows, proj_flat)


def _tc_combine(codes_b, prow, lb):
    def body(c_ref, p_ref, l_ref, o_ref):
        o_ref[...] = jnp.where(c_ref[...] >= 0, p_ref[...], l_ref[...])

    return pl.pallas_call(
        body,
        grid=(NUM_TC_BLOCKS,),
        in_specs=[
            pl.BlockSpec((TC_BLOCK, 1), lambda i: (i, 0)),
            pl.BlockSpec((TC_BLOCK, DIM), lambda i: (i, 0)),
            pl.BlockSpec((TC_BLOCK, DIM), lambda i: (i, 0)),
        ],
        out_specs=pl.BlockSpec((TC_BLOCK, DIM), lambda i: (i, 0)),
        out_shape=jax.ShapeDtypeStruct((BATCH, DIM), jnp.float32),
        compiler_params=pltpu.CompilerParams(
            dimension_semantics=("arbitrary",),
        ),
    )(codes_b, prow, lb)


def kernel(indices, entity_types, entity_data_idx, tables, W, b, lookup_table):
    indices = indices.astype(jnp.int32)
    entity_types = entity_types.astype(jnp.int32)
    entity_data_idx = entity_data_idx.astype(jnp.int32)
    codes = jnp.where(entity_types < NUM_TYPES,
                      entity_types * DATA_SIZE + entity_data_idx,
                      -1).astype(jnp.int32)

    proj = _project_tables(tables, W, b)
    proj_flat = proj.reshape(NUM_TYPES * DATA_SIZE, DIM)
    rows, codes_b, lb = _sc_route(codes, indices, lookup_table)
    prow = _sc_gather_proj(rows, proj_flat)
    return _tc_combine(codes_b.reshape(BATCH, 1), prow, lb)


# speedup vs baseline: 2.5103x; 1.3084x over previous
import functools

import jax
import jax.numpy as jnp
from jax import lax
from jax.experimental import pallas as pl
from jax.experimental.pallas import tpu as pltpu
from jax.experimental.pallas import tpu_sc as plsc

NUM_TYPES = 8
DATA_SIZE = 10000
IN_DIM = 256
DIM = 128
BATCH = 16384

NUM_WORKERS = 32
PER_WORKER = BATCH // NUM_WORKERS
CHUNK = 128
NUM_CHUNKS = PER_WORKER // CHUNK

TC_BLOCK = 2048
NUM_TC_BLOCKS = BATCH // TC_BLOCK


def _sc_gather(codes, indices, tables_flat, lookup_table):
    mesh = plsc.VectorSubcoreMesh(core_axis_name="c", subcore_axis_name="s")

    @functools.partial(
        pl.kernel,
        out_type=(
            jax.ShapeDtypeStruct((BATCH, IN_DIM), jnp.float32),
            jax.ShapeDtypeStruct((BATCH, DIM), jnp.float32),
            jax.ShapeDtypeStruct((BATCH,), jnp.int32),
        ),
        mesh=mesh,
        scratch_types=[
            pltpu.VMEM((PER_WORKER,), jnp.int32),
            pltpu.VMEM((PER_WORKER,), jnp.int32),
            pltpu.VMEM((PER_WORKER,), jnp.int32),
            pltpu.VMEM((CHUNK, IN_DIM), jnp.float32),
            pltpu.VMEM((CHUNK, IN_DIM), jnp.float32),
            pltpu.VMEM((CHUNK, DIM), jnp.float32),
            pltpu.VMEM((CHUNK, DIM), jnp.float32),
            pltpu.SemaphoreType.DMA,
            pltpu.SemaphoreType.DMA,
            pltpu.SemaphoreType.DMA,
            pltpu.SemaphoreType.DMA,
            pltpu.SemaphoreType.DMA,
        ],
    )
    def sc_kernel(codes_hbm, idx_hbm, tab_hbm, lut_hbm,
                  emb_out, lb_out, code_out,
                  idx_v, c_v, row_v, emb_v0, emb_v1, lb_v0, lb_v1,
                  csem, gsem_e, gsem_l, wsem_e, wsem_l):
        wid = lax.axis_index("s") * 2 + lax.axis_index("c")
        base = wid * PER_WORKER
        pltpu.sync_copy(idx_hbm.at[pl.ds(base, PER_WORKER)], idx_v)
        code_gets = [
            pltpu.async_copy(
                codes_hbm.at[idx_v.at[pl.ds(k * CHUNK, CHUNK)]],
                c_v.at[pl.ds(k * CHUNK, CHUNK)], csem)
            for k in range(NUM_CHUNKS)
        ]
        for cg in code_gets:
            cg.wait()

        @pl.loop(0, PER_WORKER, step=16)
        def _(i):
            c = c_v[pl.ds(i, 16)]
            pos = lax.iota(jnp.int32, 16) + (base + i)
            row_v[pl.ds(i, 16)] = jnp.where(c >= 0, c, pos)

        code_put = pltpu.async_copy(c_v, code_out.at[pl.ds(base, PER_WORKER)],
                                    csem)

        emb_bufs = (emb_v0, emb_v1)
        lb_bufs = (lb_v0, lb_v1)

        def fire(k):
            e, l = emb_bufs[k % 2], lb_bufs[k % 2]
            ge = pltpu.async_copy(
                tab_hbm.at[row_v.at[pl.ds(k * CHUNK, CHUNK)]], e, gsem_e)
            gl = pltpu.async_copy(
                lut_hbm.at[idx_v.at[pl.ds(k * CHUNK, CHUNK)]], l, gsem_l)
            return ge, gl

        pend = fire(0)
        writes = [None] * NUM_CHUNKS
        for k in range(NUM_CHUNKS):
            if k >= 1:
                writes[k - 1][0].wait()
                writes[k - 1][1].wait()
            nxt = fire(k + 1) if k + 1 < NUM_CHUNKS else None
            pend[0].wait()
            pend[1].wait()
            e, l = emb_bufs[k % 2], lb_bufs[k % 2]
            off = base + k * CHUNK
            writes[k] = (
                pltpu.async_copy(e, emb_out.at[pl.ds(off, CHUNK)], wsem_e),
                pltpu.async_copy(l, lb_out.at[pl.ds(off, CHUNK)], wsem_l),
            )
            pend = nxt
        writes[-1][0].wait()
        writes[-1][1].wait()
        code_put.wait()

    return sc_kernel(codes, indices, tables_flat, lookup_table)


def _tc_combine(codes_b, emb, lb, W, b):
    def body(c_ref, emb_ref, lb_ref, w_ref, b_ref, o_ref):
        c = c_ref[...]
        t = c // DATA_SIZE
        emb16 = emb_ref[...].astype(jnp.bfloat16)
        acc = jnp.zeros((TC_BLOCK, DIM), jnp.float32)
        for tt in range(NUM_TYPES):
            prod = jnp.dot(emb16, w_ref[tt].astype(jnp.bfloat16),
                           preferred_element_type=jnp.float32)
            acc = acc + jnp.where(t == tt, prod + b_ref[tt][None, :], 0.0)
        o_ref[...] = jnp.where(c >= 0, acc, lb_ref[...])

    return pl.pallas_call(
        body,
        grid=(NUM_TC_BLOCKS,),
        in_specs=[
            pl.BlockSpec((TC_BLOCK, 1), lambda i: (i, 0)),
            pl.BlockSpec((TC_BLOCK, IN_DIM), lambda i: (i, 0)),
            pl.BlockSpec((TC_BLOCK, DIM), lambda i: (i, 0)),
            pl.BlockSpec((NUM_TYPES, IN_DIM, DIM), lambda i: (0, 0, 0)),
            pl.BlockSpec((NUM_TYPES, DIM), lambda i: (0, 0)),
        ],
        out_specs=pl.BlockSpec((TC_BLOCK, DIM), lambda i: (i, 0)),
        out_shape=jax.ShapeDtypeStruct((BATCH, DIM), jnp.float32),
        compiler_params=pltpu.CompilerParams(
            dimension_semantics=("arbitrary",),
        ),
    )(codes_b, emb, lb, W, b)


def kernel(indices, entity_types, entity_data_idx, tables, W, b, lookup_table):
    indices = indices.astype(jnp.int32)
    entity_types = entity_types.astype(jnp.int32)
    entity_data_idx = entity_data_idx.astype(jnp.int32)
    codes = jnp.where(entity_types < NUM_TYPES,
                      entity_types * DATA_SIZE + entity_data_idx,
                      -1).astype(jnp.int32)
    tables_flat = tables.reshape(NUM_TYPES * DATA_SIZE, IN_DIM)

    emb, lb, codes_b = _sc_gather(codes, indices, tables_flat, lookup_table)
    return _tc_combine(codes_b.reshape(BATCH, 1), emb, lb, W, b)


# speedup vs baseline: 2.5359x; 1.0102x over previous
import functools

import jax
import jax.numpy as jnp
from jax import lax
from jax.experimental import pallas as pl
from jax.experimental.pallas import tpu as pltpu
from jax.experimental.pallas import tpu_sc as plsc

NUM_TYPES = 8
DATA_SIZE = 10000
IN_DIM = 256
DIM = 128
BATCH = 16384

NUM_WORKERS = 32
PER_WORKER = BATCH // NUM_WORKERS
CHUNK = 128
NUM_CHUNKS = PER_WORKER // CHUNK

TC_BLOCK = 2048
NUM_TC_BLOCKS = BATCH // TC_BLOCK


def _sc_gather(codes, indices, tables_flat, lookup_table):
    mesh = plsc.VectorSubcoreMesh(core_axis_name="c", subcore_axis_name="s")

    @functools.partial(
        pl.kernel,
        out_type=(
            jax.ShapeDtypeStruct((BATCH, IN_DIM), jnp.float32),
            jax.ShapeDtypeStruct((BATCH, DIM), jnp.float32),
            jax.ShapeDtypeStruct((BATCH,), jnp.int32),
        ),
        mesh=mesh,
        scratch_types=[
            pltpu.VMEM((PER_WORKER,), jnp.int32),
            pltpu.VMEM((PER_WORKER,), jnp.int32),
            pltpu.VMEM((PER_WORKER,), jnp.int32),
            pltpu.VMEM((CHUNK, IN_DIM), jnp.float32),
            pltpu.VMEM((CHUNK, IN_DIM), jnp.float32),
            pltpu.VMEM((CHUNK, DIM), jnp.float32),
            pltpu.VMEM((CHUNK, DIM), jnp.float32),
            pltpu.SemaphoreType.DMA,
            pltpu.SemaphoreType.DMA,
            pltpu.SemaphoreType.DMA,
            pltpu.SemaphoreType.DMA,
            pltpu.SemaphoreType.DMA,
        ],
    )
    def sc_kernel(codes_hbm, idx_hbm, tab_hbm, lut_hbm,
                  emb_out, lb_out, code_out,
                  idx_v, c_v, row_v, emb_v0, emb_v1, lb_v0, lb_v1,
                  csem, gsem_e, gsem_l, wsem_e, wsem_l):
        wid = lax.axis_index("s") * 2 + lax.axis_index("c")
        base = wid * PER_WORKER
        pltpu.sync_copy(idx_hbm.at[pl.ds(base, PER_WORKER)], idx_v)
        code_gets = [
            pltpu.async_copy(
                codes_hbm.at[idx_v.at[pl.ds(k * CHUNK, CHUNK)]],
                c_v.at[pl.ds(k * CHUNK, CHUNK)], csem)
            for k in range(NUM_CHUNKS)
        ]
        for cg in code_gets:
            cg.wait()

        @pl.loop(0, PER_WORKER, step=16)
        def _(i):
            c = c_v[pl.ds(i, 16)]
            pos = lax.iota(jnp.int32, 16) + (base + i)
            row_v[pl.ds(i, 16)] = jnp.where(c >= 0, c, pos)

        code_put = pltpu.async_copy(c_v, code_out.at[pl.ds(base, PER_WORKER)],
                                    csem)

        emb_bufs = (emb_v0, emb_v1)
        lb_bufs = (lb_v0, lb_v1)

        def fire(k):
            e, l = emb_bufs[k % 2], lb_bufs[k % 2]
            ge = pltpu.async_copy(
                tab_hbm.at[row_v.at[pl.ds(k * CHUNK, CHUNK)]], e, gsem_e)
            gl = pltpu.async_copy(
                lut_hbm.at[idx_v.at[pl.ds(k * CHUNK, CHUNK)]], l, gsem_l)
            return ge, gl

        pend = fire(0)
        writes = [None] * NUM_CHUNKS
        for k in range(NUM_CHUNKS):
            if k >= 1:
                writes[k - 1][0].wait()
                writes[k - 1][1].wait()
            nxt = fire(k + 1) if k + 1 < NUM_CHUNKS else None
            pend[0].wait()
            pend[1].wait()
            e, l = emb_bufs[k % 2], lb_bufs[k % 2]
            off = base + k * CHUNK
            writes[k] = (
                pltpu.async_copy(e, emb_out.at[pl.ds(off, CHUNK)], wsem_e),
                pltpu.async_copy(l, lb_out.at[pl.ds(off, CHUNK)], wsem_l),
            )
            pend = nxt
        writes[-1][0].wait()
        writes[-1][1].wait()
        code_put.wait()

    return sc_kernel(codes, indices, tables_flat, lookup_table)


def _tc_combine(codes_b, emb, lb, W, b):
    def body(c_ref, emb_ref, lb_ref, w_ref, b_ref, o_ref):
        c = c_ref[...]
        t = c // DATA_SIZE
        emb16 = emb_ref[...].astype(jnp.bfloat16)
        acc = jnp.zeros((TC_BLOCK, DIM), jnp.bfloat16)
        for tt in range(NUM_TYPES):
            prod = jnp.dot(emb16, w_ref[tt].astype(jnp.bfloat16),
                           preferred_element_type=jnp.float32)
            prod16 = (prod + b_ref[tt][None, :]).astype(jnp.bfloat16)
            acc = jnp.where(t == tt, prod16, acc)
        o_ref[...] = jnp.where(c >= 0, acc.astype(jnp.float32), lb_ref[...])

    return pl.pallas_call(
        body,
        grid=(NUM_TC_BLOCKS,),
        in_specs=[
            pl.BlockSpec((TC_BLOCK, 1), lambda i: (i, 0)),
            pl.BlockSpec((TC_BLOCK, IN_DIM), lambda i: (i, 0)),
            pl.BlockSpec((TC_BLOCK, DIM), lambda i: (i, 0)),
            pl.BlockSpec((NUM_TYPES, IN_DIM, DIM), lambda i: (0, 0, 0)),
            pl.BlockSpec((NUM_TYPES, DIM), lambda i: (0, 0)),
        ],
        out_specs=pl.BlockSpec((TC_BLOCK, DIM), lambda i: (i, 0)),
        out_shape=jax.ShapeDtypeStruct((BATCH, DIM), jnp.float32),
        compiler_params=pltpu.CompilerParams(
            dimension_semantics=("arbitrary",),
        ),
    )(codes_b, emb, lb, W, b)


def kernel(indices, entity_types, entity_data_idx, tables, W, b, lookup_table):
    indices = indices.astype(jnp.int32)
    entity_types = entity_types.astype(jnp.int32)
    entity_data_idx = entity_data_idx.astype(jnp.int32)
    codes = jnp.where(entity_types < NUM_TYPES,
                      entity_types * DATA_SIZE + entity_data_idx,
                      -1).astype(jnp.int32)
    tables_flat = tables.reshape(NUM_TYPES * DATA_SIZE, IN_DIM)

    emb, lb, codes_b = _sc_gather(codes, indices, tables_flat, lookup_table)
    return _tc_combine(codes_b.reshape(BATCH, 1), emb, lb, W, b)


# speedup vs baseline: 3.0578x; 1.2058x over previous
import functools

import jax
import jax.numpy as jnp
from jax import lax
from jax.experimental import pallas as pl
from jax.experimental.pallas import tpu as pltpu
from jax.experimental.pallas import tpu_sc as plsc

NUM_TYPES = 8
DATA_SIZE = 10000
IN_DIM = 256
DIM = 128
BATCH = 16384

NUM_WORKERS = 32
PER_WORKER = BATCH // NUM_WORKERS
CHUNK = 128
NUM_CHUNKS = PER_WORKER // CHUNK

TC_BLOCK = 2048
NUM_TC_BLOCKS = BATCH // TC_BLOCK


def _sc_gather(codes, indices, tables_flat, lookup_table):
    mesh = plsc.VectorSubcoreMesh(core_axis_name="c", subcore_axis_name="s")

    @functools.partial(
        pl.kernel,
        out_type=(
            jax.ShapeDtypeStruct((BATCH, IN_DIM), jnp.float32),
            jax.ShapeDtypeStruct((BATCH, DIM), jnp.float32),
            jax.ShapeDtypeStruct((BATCH,), jnp.int32),
        ),
        mesh=mesh,
        scratch_types=[
            pltpu.VMEM((PER_WORKER,), jnp.int32),
            pltpu.VMEM((PER_WORKER,), jnp.int32),
            pltpu.VMEM((PER_WORKER,), jnp.int32),
            pltpu.VMEM((CHUNK, IN_DIM), jnp.float32),
            pltpu.VMEM((CHUNK, IN_DIM), jnp.float32),
            pltpu.VMEM((CHUNK, DIM), jnp.float32),
            pltpu.VMEM((CHUNK, DIM), jnp.float32),
            pltpu.SemaphoreType.DMA,
            pltpu.SemaphoreType.DMA,
            pltpu.SemaphoreType.DMA,
            pltpu.SemaphoreType.DMA,
            pltpu.SemaphoreType.DMA,
        ],
    )
    def sc_kernel(codes_hbm, idx_hbm, tab_hbm, lut_hbm,
                  emb_out, lb_out, code_out,
                  idx_v, c_v, row_v, emb_v0, emb_v1, lb_v0, lb_v1,
                  csem, gsem_e, gsem_l, wsem_e, wsem_l):
        wid = lax.axis_index("s") * 2 + lax.axis_index("c")
        base = wid * PER_WORKER
        pltpu.sync_copy(idx_hbm.at[pl.ds(base, PER_WORKER)], idx_v)
        code_gets = [
            pltpu.async_copy(
                codes_hbm.at[idx_v.at[pl.ds(k * CHUNK, CHUNK)]],
                c_v.at[pl.ds(k * CHUNK, CHUNK)], csem)
            for k in range(NUM_CHUNKS)
        ]
        for cg in code_gets:
            cg.wait()

        @pl.loop(0, PER_WORKER, step=16)
        def _(i):
            c = c_v[pl.ds(i, 16)]
            pos = lax.iota(jnp.int32, 16) + (base + i)
            row_v[pl.ds(i, 16)] = jnp.where(c >= 0, c, pos)

        code_put = pltpu.async_copy(c_v, code_out.at[pl.ds(base, PER_WORKER)],
                                    csem)

        emb_bufs = (emb_v0, emb_v1)
        lb_bufs = (lb_v0, lb_v1)

        def fire(k):
            e, l = emb_bufs[k % 2], lb_bufs[k % 2]
            ge = pltpu.async_copy(
                tab_hbm.at[row_v.at[pl.ds(k * CHUNK, CHUNK)]], e, gsem_e)
            gl = pltpu.async_copy(
                lut_hbm.at[idx_v.at[pl.ds(k * CHUNK, CHUNK)]], l, gsem_l)
            return ge, gl

        pend = fire(0)
        writes = [None] * NUM_CHUNKS
        for k in range(NUM_CHUNKS):
            if k >= 1:
                writes[k - 1][0].wait()
                writes[k - 1][1].wait()
            nxt = fire(k + 1) if k + 1 < NUM_CHUNKS else None
            pend[0].wait()
            pend[1].wait()
            e, l = emb_bufs[k % 2], lb_bufs[k % 2]
            off = base + k * CHUNK
            writes[k] = (
                pltpu.async_copy(e, emb_out.at[pl.ds(off, CHUNK)], wsem_e),
                pltpu.async_copy(l, lb_out.at[pl.ds(off, CHUNK)], wsem_l),
            )
            pend = nxt
        writes[-1][0].wait()
        writes[-1][1].wait()
        code_put.wait()

    return sc_kernel(codes, indices, tables_flat, lookup_table)


def _tc_combine(codes_b, emb, lb, W, b):
    def body(c_ref, emb_ref, lb_ref, w_ref, b_ref, o_ref):
        crow = c_ref[0]
        srow = jnp.where(crow >= 0, crow // DATA_SIZE, NUM_TYPES)
        scol = srow.reshape(TC_BLOCK, 1).astype(jnp.int16)
        sfull = scol + jnp.zeros((TC_BLOCK, DIM), jnp.int16)
        emb16 = emb_ref[...].astype(jnp.bfloat16)
        acc = lb_ref[...].astype(jnp.bfloat16)
        for tt in range(NUM_TYPES):
            prod = jnp.dot(emb16, w_ref[tt].astype(jnp.bfloat16),
                           preferred_element_type=jnp.float32)
            prod16 = prod.astype(jnp.bfloat16) + b_ref[tt].astype(jnp.bfloat16)[None, :]
            acc = jnp.where(sfull == jnp.int16(tt), prod16, acc)
        o_ref[...] = acc.astype(jnp.float32)

    return pl.pallas_call(
        body,
        grid=(NUM_TC_BLOCKS,),
        in_specs=[
            pl.BlockSpec((1, 1, TC_BLOCK), lambda i: (i, 0, 0)),
            pl.BlockSpec((TC_BLOCK, IN_DIM), lambda i: (i, 0)),
            pl.BlockSpec((TC_BLOCK, DIM), lambda i: (i, 0)),
            pl.BlockSpec((NUM_TYPES, IN_DIM, DIM), lambda i: (0, 0, 0)),
            pl.BlockSpec((NUM_TYPES, DIM), lambda i: (0, 0)),
        ],
        out_specs=pl.BlockSpec((TC_BLOCK, DIM), lambda i: (i, 0)),
        out_shape=jax.ShapeDtypeStruct((BATCH, DIM), jnp.float32),
        compiler_params=pltpu.CompilerParams(
            dimension_semantics=("arbitrary",),
        ),
    )(codes_b, emb, lb, W, b)


def kernel(indices, entity_types, entity_data_idx, tables, W, b, lookup_table):
    indices = indices.astype(jnp.int32)
    entity_types = entity_types.astype(jnp.int32)
    entity_data_idx = entity_data_idx.astype(jnp.int32)
    codes = jnp.where(entity_types < NUM_TYPES,
                      entity_types * DATA_SIZE + entity_data_idx,
                      -1).astype(jnp.int32)
    tables_flat = tables.reshape(NUM_TYPES * DATA_SIZE, IN_DIM)

    emb, lb, codes_b = _sc_gather(codes, indices, tables_flat, lookup_table)
    return _tc_combine(codes_b.reshape(NUM_TC_BLOCKS, 1, TC_BLOCK),
                       emb, lb, W, b)
